# baseline (speedup 1.0000x reference)
import functools

import jax
import jax.numpy as jnp
from jax import lax
from jax.experimental import pallas as pl
from jax.experimental.pallas import tpu as pltpu
from jax.experimental.pallas import tpu_sc as plsc

N = 10000
E = 320000
D = 128
H = 128
C = 64

_BN = 1000


def _mid_body(p0_ref, p1_ref, w1_ref, b1_ref, w2_ref, o_ref):
    agg1 = jnp.dot(p0_ref[...] + p1_ref[...], w1_ref[...],
                   preferred_element_type=jnp.float32)
    h = jnp.maximum(agg1 + b1_ref[...], 0.0)
    o_ref[...] = jnp.dot(h, w2_ref[...], preferred_element_type=jnp.float32)


def _mid(p0, p1, W1, b1, W2):
    return pl.pallas_call(
        _mid_body,
        out_shape=jax.ShapeDtypeStruct((N, C), jnp.float32),
    )(p0, p1, W1, b1.reshape(1, H), W2)


def _softmax_body(p0_ref, p1_ref, b2_ref, o_ref):
    x = p0_ref[...] + p1_ref[...] + b2_ref[...]
    m = jnp.max(x, axis=1, keepdims=True)
    e = jnp.exp(x - m)
    o_ref[...] = e / jnp.sum(e, axis=1, keepdims=True)


def _softmax(p0, p1, b2):
    return pl.pallas_call(
        _softmax_body,
        out_shape=jax.ShapeDtypeStruct((N, C), jnp.float32),
    )(p0, p1, b2.reshape(1, C))


_NC = 2
_NS = 16
_B = 80
_ET = E // (_NC * _NS)
_CPT = _ET // _B
_NPK = 4
_NRW = 3
_ZR = 80
_NRB = N // _ZR


def _make_segsum(Hd):
    mesh = plsc.VectorSubcoreMesh(core_axis_name="c", subcore_axis_name="s")

    @functools.partial(
        pl.kernel,
        out_type=jax.ShapeDtypeStruct((_NC, N, Hd), jnp.float32),
        mesh=mesh,
        scratch_types=(
            [pltpu.VMEM((2, _B), jnp.int32) for _ in range(_NPK)]
            + [pltpu.VMEM((_B,), jnp.float32) for _ in range(_NPK)]
            + [pltpu.VMEM((_B, Hd), jnp.float32) for _ in range(_NRW)]
            + [pltpu.VMEM((_ZR, Hd), jnp.float32)]
            + [pltpu.VMEM_SHARED((N, Hd), jnp.float32)]
            + [pltpu.SemaphoreType.DMA for _ in range(_NPK + 2 * _NRW)]
        ),
        compiler_params=pltpu.CompilerParams(needs_layout_passes=False,
                                             use_tc_tiling_on_sc=False),
    )
    def seg(sup_hbm, ei_hbm, w_hbm, out_hbm,
            pk0, pk1, pk2, pk3, wv0, wv1, wv2, wv3,
            rw0, rw1, rw2, zbuf, accum, *sems):
        pk = [pk0, pk1, pk2, pk3]
        wv = [wv0, wv1, wv2, wv3]
        rw = [rw0, rw1, rw2]
        semA = sems[0:_NPK]
        semG = sems[_NPK:_NPK + _NRW]
        semD = sems[_NPK + _NRW:_NPK + 2 * _NRW]
        cid = lax.axis_index("c")
        sid = lax.axis_index("s")
        be = (cid * _NS + sid) * _ET

        def issueA(j, bp):
            base = pl.multiple_of(be + j * _B, 8)
            pltpu.async_copy(ei_hbm.at[:, pl.ds(base, _B)], pk[bp], semA[bp])
            pltpu.async_copy(w_hbm.at[pl.ds(base, _B)], wv[bp], semA[bp])

        def issueB(j, bp, br):
            pltpu.make_async_copy(ei_hbm.at[:, pl.ds(0, _B)], pk[bp],
                                  semA[bp]).wait()
            pltpu.make_async_copy(w_hbm.at[pl.ds(0, _B)], wv[bp],
                                  semA[bp]).wait()
            pltpu.async_copy(sup_hbm.at[pk[bp].at[1]], rw[br], semG[br])

        def waitG(bp, br):
            pltpu.make_async_copy(sup_hbm.at[pk[bp].at[1]], rw[br],
                                  semG[br]).wait()

        def scale(bp, br):
            def _grp(q, _):
                e0 = q * 16
                w16 = wv[bp][pl.ds(e0, 16)]
                dn = lax.GatherDimensionNumbers(
                    offset_dims=(), collapsed_slice_dims=(0,),
                    start_index_map=(0,))
                for u in range(16):
                    e = e0 + u
                    wb = lax.gather(
                        w16, jnp.full((16, 1), u, jnp.int32), dn, (1,),
                        mode=lax.GatherScatterMode.PROMISE_IN_BOUNDS)
                    for hh in range(Hd // 16):
                        sl = pl.ds(hh * 16, 16)
                        rw[br][e, sl] = rw[br][e, sl] * wb
                return _
            lax.fori_loop(0, _B // 16, _grp, None)

        def issueD(bp, br):
            pltpu.async_copy(rw[br], accum.at[pk[bp].at[0]], semD[br],
                             add=True)

        def waitD(bp, br):
            pltpu.make_async_copy(rw[br], accum.at[pk[bp].at[0]],
                                  semD[br]).wait()

        def body(j, bp, br):
            waitG(bp, br)
            scale(bp, br)
            issueD(bp, br)

        def _zb(r, _):
            for hh in range(Hd // 16):
                zbuf[r, pl.ds(hh * 16, 16)] = jnp.zeros((16,), jnp.float32)
            return _
        lax.fori_loop(0, _ZR, _zb, None)
        nrb = (_NRB - sid + _NS - 1) // _NS

        def _zi(m, _):
            r0 = pl.multiple_of((sid + _NS * m) * _ZR, 8)
            pltpu.async_copy(zbuf, accum.at[pl.ds(r0, _ZR)], semA[0])
            return _
        lax.fori_loop(0, nrb, _zi, None)

        def _zw(m, _):
            pltpu.make_async_copy(zbuf, accum.at[pl.ds(0, _ZR)],
                                  semA[0]).wait()
            return _
        lax.fori_loop(0, nrb, _zw, None)
        plsc.subcore_barrier()

        issueA(0, 0)
        issueA(1, 1)
        issueB(0, 0, 0)
        for j in (0, 1):
            issueA(j + 2, (j + 2) % _NPK)
            issueB(j + 1, (j + 1) % _NPK, (j + 1) % _NRW)
            body(j, j % _NPK, j % _NRW)

        def _main(g, _):
            j0 = 2 + 12 * g
            for k in range(12):
                j = j0 + k
                p2, r2 = (2 + k + 2) % _NPK, (2 + k + 2) % _NRW
                waitD(p2, (2 + k - 2) % _NRW)
                issueA(j + 2, p2)
                issueB(j + 1, (2 + k + 1) % _NPK, (2 + k + 1) % _NRW)
                body(j, (2 + k) % _NPK, (2 + k) % _NRW)
            return _
        lax.fori_loop(0, (_CPT - 5) // 12, _main, None)

        for j in range(_CPT - 3, _CPT):
            waitD((j - 2) % _NPK, (j - 2) % _NRW)
            if j + 2 < _CPT:
                issueA(j + 2, (j + 2) % _NPK)
            if j + 1 < _CPT:
                issueB(j + 1, (j + 1) % _NPK, (j + 1) % _NRW)
            body(j, j % _NPK, j % _NRW)
        waitD((_CPT - 2) % _NPK, (_CPT - 2) % _NRW)
        waitD((_CPT - 1) % _NPK, (_CPT - 1) % _NRW)
        plsc.subcore_barrier()

        def _wo(m, _):
            r0 = pl.multiple_of((sid + _NS * m) * _ZR, 8)
            pltpu.async_copy(accum.at[pl.ds(r0, _ZR)],
                             out_hbm.at[cid, pl.ds(r0, _ZR)], semA[0])
            return _
        lax.fori_loop(0, nrb, _wo, None)

        def _ww(m, _):
            pltpu.make_async_copy(accum.at[pl.ds(0, _ZR)],
                                  out_hbm.at[cid, pl.ds(0, _ZR)],
                                  semA[0]).wait()
            return _
        lax.fori_loop(0, nrb, _ww, None)

    return seg


_segsum_h = _make_segsum(H)
_segsum_c = _make_segsum(C)


def _segsum(support, ei, w, Hd):
    f = _segsum_h if Hd == H else _segsum_c
    out = f(support, ei, w)
    return out[0], out[1]


@jax.jit
def kernel(feat, view_edge_index, view_edge_weight, W1, b1, W2, b2):
    a0, a1 = _segsum(feat, view_edge_index, view_edge_weight, H)
    support2 = _mid(a0, a1, W1, b1, W2)
    g0, g1 = _segsum(support2, view_edge_index, view_edge_weight, C)
    return _softmax(g0, g1, b2)


# speedup vs baseline: 1.3342x; 1.3342x over previous
import functools

import jax
import jax.numpy as jnp
from jax import lax
from jax.experimental import pallas as pl
from jax.experimental.pallas import tpu as pltpu
from jax.experimental.pallas import tpu_sc as plsc

N = 10000
E = 320000
D = 128
H = 128
C = 64

_BN = 1000


def _mid_body(p0_ref, p1_ref, w1_ref, b1_ref, w2_ref, o_ref):
    agg1 = jnp.dot(p0_ref[...] + p1_ref[...], w1_ref[...],
                   preferred_element_type=jnp.float32)
    h = jnp.maximum(agg1 + b1_ref[...], 0.0)
    o_ref[...] = jnp.dot(h, w2_ref[...], preferred_element_type=jnp.float32)


def _mid(p0, p1, W1, b1, W2):
    return pl.pallas_call(
        _mid_body,
        out_shape=jax.ShapeDtypeStruct((N, C), jnp.float32),
    )(p0, p1, W1, b1.reshape(1, H), W2)


def _softmax_body(p0_ref, p1_ref, b2_ref, o_ref):
    x = p0_ref[...] + p1_ref[...] + b2_ref[...]
    m = jnp.max(x, axis=1, keepdims=True)
    e = jnp.exp(x - m)
    o_ref[...] = e / jnp.sum(e, axis=1, keepdims=True)


def _softmax(p0, p1, b2):
    return pl.pallas_call(
        _softmax_body,
        out_shape=jax.ShapeDtypeStruct((N, C), jnp.float32),
    )(p0, p1, b2.reshape(1, C))


_NC = 2
_NS = 16
_B = 80
_ET = E // (_NC * _NS)
_CPT = _ET // _B
_NPK = 4
_NRW = 3
_ZR = 80
_NRB = N // _ZR


def _make_segsum(Hd):
    mesh = plsc.VectorSubcoreMesh(core_axis_name="c", subcore_axis_name="s")

    @functools.partial(
        pl.kernel,
        out_type=jax.ShapeDtypeStruct((_NC, N, Hd), jnp.float32),
        mesh=mesh,
        scratch_types=(
            [pltpu.VMEM((2, _B), jnp.int32) for _ in range(_NPK)]
            + [pltpu.VMEM((_B,), jnp.float32) for _ in range(_NPK)]
            + [pltpu.VMEM((_B, Hd), jnp.float32) for _ in range(_NRW)]
            + [pltpu.VMEM((_ZR, Hd), jnp.float32)]
            + [pltpu.VMEM_SHARED((N, Hd), jnp.float32)]
            + [pltpu.SemaphoreType.DMA for _ in range(_NPK + 2 * _NRW)]
        ),
        compiler_params=pltpu.CompilerParams(needs_layout_passes=False,
                                             use_tc_tiling_on_sc=False),
    )
    def seg(sup_hbm, ei_hbm, w_hbm, out_hbm,
            pk0, pk1, pk2, pk3, wv0, wv1, wv2, wv3,
            rw0, rw1, rw2, zbuf, accum, *sems):
        pk = [pk0, pk1, pk2, pk3]
        wv = [wv0, wv1, wv2, wv3]
        rw = [rw0, rw1, rw2]
        semA = sems[0:_NPK]
        semG = sems[_NPK:_NPK + _NRW]
        semD = sems[_NPK + _NRW:_NPK + 2 * _NRW]
        cid = lax.axis_index("c")
        sid = lax.axis_index("s")
        be = (cid * _NS + sid) * _ET

        def issueA(j, bp):
            base = pl.multiple_of(be + j * _B, 8)
            pltpu.async_copy(ei_hbm.at[:, pl.ds(base, _B)], pk[bp], semA[bp])
            pltpu.async_copy(w_hbm.at[pl.ds(base, _B)], wv[bp], semA[bp])

        def issueB(j, bp, br):
            pltpu.make_async_copy(ei_hbm.at[:, pl.ds(0, _B)], pk[bp],
                                  semA[bp]).wait()
            pltpu.make_async_copy(w_hbm.at[pl.ds(0, _B)], wv[bp],
                                  semA[bp]).wait()
            pltpu.async_copy(sup_hbm.at[pk[bp].at[1]], rw[br], semG[br])

        def waitG(bp, br):
            pltpu.make_async_copy(sup_hbm.at[pk[bp].at[1]], rw[br],
                                  semG[br]).wait()

        def scale(bp, br):
            @plsc.parallel_loop(0, _B, 4)
            def _grp(e0):
                for u in range(4):
                    e = e0 + u
                    wb = plsc.load_gather(
                        wv[bp], [jnp.full((16,), e, jnp.int32)])
                    for hh in range(Hd // 16):
                        sl = pl.ds(hh * 16, 16)
                        rw[br][e, sl] = rw[br][e, sl] * wb

        def issueD(bp, br):
            pltpu.async_copy(rw[br], accum.at[pk[bp].at[0]], semD[br],
                             add=True)

        def waitD(bp, br):
            pltpu.make_async_copy(rw[br], accum.at[pk[bp].at[0]],
                                  semD[br]).wait()

        def body(j, bp, br):
            waitG(bp, br)
            scale(bp, br)
            issueD(bp, br)

        def _zb(r, _):
            for hh in range(Hd // 16):
                zbuf[r, pl.ds(hh * 16, 16)] = jnp.zeros((16,), jnp.float32)
            return _
        lax.fori_loop(0, _ZR, _zb, None)
        nrb = (_NRB - sid + _NS - 1) // _NS

        def _zi(m, _):
            r0 = pl.multiple_of((sid + _NS * m) * _ZR, 8)
            pltpu.async_copy(zbuf, accum.at[pl.ds(r0, _ZR)], semA[0])
            return _
        lax.fori_loop(0, nrb, _zi, None)

        def _zw(m, _):
            pltpu.make_async_copy(zbuf, accum.at[pl.ds(0, _ZR)],
                                  semA[0]).wait()
            return _
        lax.fori_loop(0, nrb, _zw, None)
        plsc.subcore_barrier()

        issueA(0, 0)
        issueA(1, 1)
        issueB(0, 0, 0)
        for j in (0, 1):
            issueA(j + 2, (j + 2) % _NPK)
            issueB(j + 1, (j + 1) % _NPK, (j + 1) % _NRW)
            body(j, j % _NPK, j % _NRW)

        def _main(g, _):
            j0 = 2 + 12 * g
            for k in range(12):
                j = j0 + k
                p2, r2 = (2 + k + 2) % _NPK, (2 + k + 2) % _NRW
                waitD(p2, (2 + k - 2) % _NRW)
                issueA(j + 2, p2)
                issueB(j + 1, (2 + k + 1) % _NPK, (2 + k + 1) % _NRW)
                body(j, (2 + k) % _NPK, (2 + k) % _NRW)
            return _
        lax.fori_loop(0, (_CPT - 5) // 12, _main, None)

        for j in range(_CPT - 3, _CPT):
            waitD((j - 2) % _NPK, (j - 2) % _NRW)
            if j + 2 < _CPT:
                issueA(j + 2, (j + 2) % _NPK)
            if j + 1 < _CPT:
                issueB(j + 1, (j + 1) % _NPK, (j + 1) % _NRW)
            body(j, j % _NPK, j % _NRW)
        waitD((_CPT - 2) % _NPK, (_CPT - 2) % _NRW)
        waitD((_CPT - 1) % _NPK, (_CPT - 1) % _NRW)
        plsc.subcore_barrier()

        def _wo(m, _):
            r0 = pl.multiple_of((sid + _NS * m) * _ZR, 8)
            pltpu.async_copy(accum.at[pl.ds(r0, _ZR)],
                             out_hbm.at[cid, pl.ds(r0, _ZR)], semA[0])
            return _
        lax.fori_loop(0, nrb, _wo, None)

        def _ww(m, _):
            pltpu.make_async_copy(accum.at[pl.ds(0, _ZR)],
                                  out_hbm.at[cid, pl.ds(0, _ZR)],
                                  semA[0]).wait()
            return _
        lax.fori_loop(0, nrb, _ww, None)

    return seg


_segsum_h = _make_segsum(H)
_segsum_c = _make_segsum(C)


def _segsum(support, ei, w, Hd):
    f = _segsum_h if Hd == H else _segsum_c
    out = f(support, ei, w)
    return out[0], out[1]


@jax.jit
def kernel(feat, view_edge_index, view_edge_weight, W1, b1, W2, b2):
    a0, a1 = _segsum(feat, view_edge_index, view_edge_weight, H)
    support2 = _mid(a0, a1, W1, b1, W2)
    g0, g1 = _segsum(support2, view_edge_index, view_edge_weight, C)
    return _softmax(g0, g1, b2)
